# Initial kernel scaffold; baseline (speedup 1.0000x reference)
import jax
import jax.numpy as jnp
from jax.experimental import pallas as pl


def kernel(params, emb_user, emb_post, ei_viewed, ei_rev_viewed, ei_liked, ei_rev_liked):
    raise NotImplementedError("write your pallas kernel here")


# speedup vs baseline: 46.3619x; 46.3619x over previous
import functools

import jax
import jax.numpy as jnp
from jax import lax
from jax.experimental import pallas as pl
from jax.experimental.pallas import tpu as pltpu
from jax.experimental.pallas import tpu_sc as plsc

NUM_USERS = 631
NUM_POSTS = 1210
D = 64
NKEYS = NUM_USERS * NUM_POSTS
HIST = 763648
SLICE = HIST // 16
NSUB = 16
BLK = 1024
ZCHUNK = 8192
_CHUNKS = [(o, min(ZCHUNK, SLICE - o)) for o in range(0, SLICE, ZCHUNK)]


def _ceil_to(x, m):
    return (x + m - 1) // m * m


def _build_hists(sv, dv, sl_, dl_, srv, drv, srl, drl):
    epad_a = sv.shape[0]
    epad_b = sl_.shape[0]

    mesh = plsc.VectorSubcoreMesh(core_axis_name="c", subcore_axis_name="s")

    @functools.partial(
        pl.kernel,
        mesh=mesh,
        out_type=[jax.ShapeDtypeStruct((HIST,), jnp.float32)] * 4,
        scratch_types=[
            pltpu.VMEM((BLK,), jnp.int32),
            pltpu.VMEM((BLK,), jnp.int32),
            pltpu.VMEM((BLK // 128, 128), jnp.int32),
            pltpu.VMEM((128,), jnp.float32),
            pltpu.VMEM((ZCHUNK,), jnp.float32),
            pltpu.VMEM_SHARED((HIST,), jnp.float32),
            pltpu.VMEM_SHARED((HIST,), jnp.float32),
        ],
    )
    def hist_kernel(sv_h, dv_h, sl_h, dl_h, srv_h, drv_h, srl_h, drl_h,
                    out_v, out_l, out_rv, out_rl,
                    src_buf, dst_buf, keys, ones, zeros, hist_a, hist_b):
        core = lax.axis_index("c")
        sid = lax.axis_index("s")
        base = sid * SLICE

        @pl.loop(0, 128, step=16)
        def _(i):
            ones[pl.ds(i, 16)] = jnp.ones((16,), jnp.float32)

        @pl.loop(0, ZCHUNK, step=16)
        def _(i):
            zeros[pl.ds(i, 16)] = jnp.zeros((16,), jnp.float32)

        for off, n in _CHUNKS:
            pltpu.sync_copy(zeros.at[pl.ds(0, n)], hist_a.at[pl.ds(base + off, n)])
            pltpu.sync_copy(zeros.at[pl.ds(0, n)], hist_b.at[pl.ds(base + off, n)])
        plsc.subcore_barrier()

        def process(src_hbm, dst_hbm, epad, mult, hist):
            chunk = epad // NSUB
            ebase = sid * chunk

            @pl.loop(0, chunk, step=BLK)
            def _(b):
                pltpu.sync_copy(src_hbm.at[pl.ds(ebase + b, BLK)], src_buf)
                pltpu.sync_copy(dst_hbm.at[pl.ds(ebase + b, BLK)], dst_buf)
                for j in range(BLK // 128):
                    krow = keys.at[j]

                    @pl.loop(0, 128, step=16)
                    def _(i):
                        s16 = src_buf[pl.ds(j * 128 + i, 16)]
                        d16 = dst_buf[pl.ds(j * 128 + i, 16)]
                        krow[pl.ds(i, 16)] = d16 * mult + s16

                    pltpu.sync_copy(ones, hist.at[krow], add=True)

        @pl.when(core == 0)
        def _():
            process(sv_h, dv_h, epad_a, NUM_USERS, hist_a)
            process(sl_h, dl_h, epad_b, NUM_USERS, hist_b)

        @pl.when(core == 1)
        def _():
            process(srv_h, drv_h, epad_a, NUM_POSTS, hist_a)
            process(srl_h, drl_h, epad_b, NUM_POSTS, hist_b)

        plsc.subcore_barrier()

        def readout(hist, out):
            for off, n in _CHUNKS:
                pltpu.sync_copy(hist.at[pl.ds(base + off, n)], zeros.at[pl.ds(0, n)])
                pltpu.sync_copy(zeros.at[pl.ds(0, n)], out.at[pl.ds(base + off, n)])

        @pl.when(core == 0)
        def _():
            readout(hist_a, out_v)
            readout(hist_b, out_l)

        @pl.when(core == 1)
        def _():
            readout(hist_a, out_rv)
            readout(hist_b, out_rl)

    return hist_kernel(sv, dv, sl_, dl_, srv, drv, srl, drl)


def _dense_body(cv, cl, crv, crl, xu, xp, wl, wr, bl, u_out, p_out):
    Cv = cv[...]
    Cl = cl[...]
    Crv = crv[...]
    Crl = crl[...]
    xu_ = xu[...]
    xp_ = xp[...]

    inv_v = 1.0 / jnp.maximum(jnp.sum(Cv, axis=1, keepdims=True), 1.0)
    inv_l = 1.0 / jnp.maximum(jnp.sum(Cl, axis=1, keepdims=True), 1.0)
    inv_rv = 1.0 / jnp.maximum(jnp.sum(Crv, axis=1, keepdims=True), 1.0)
    inv_rl = 1.0 / jnp.maximum(jnp.sum(Crl, axis=1, keepdims=True), 1.0)

    def conv(C, inv, xs, xd, i):
        mean = jnp.dot(C, xs, preferred_element_type=jnp.float32) * inv
        return (jnp.dot(mean, wl[i], preferred_element_type=jnp.float32)
                + bl[i][None, :]
                + jnp.dot(xd, wr[i], preferred_element_type=jnp.float32))

    p1 = jax.nn.relu(conv(Cv, inv_v, xu_, xp_, 0) + conv(Cl, inv_l, xu_, xp_, 1))
    u1 = jax.nn.relu(conv(Crv, inv_rv, xp_, xu_, 2) + conv(Crl, inv_rl, xp_, xu_, 3))
    p2 = conv(Cv, inv_v, u1, p1, 4) + conv(Cl, inv_l, u1, p1, 5)
    u2 = conv(Crv, inv_rv, p1, u1, 6) + conv(Crl, inv_rl, p1, u1, 7)
    u_out[...] = u2
    p_out[...] = p2


def _dense(Cv, Cl, Crv, Crl, xu, xp, wl, wr, bl, interpret=False):
    return pl.pallas_call(
        _dense_body,
        out_shape=[
            jax.ShapeDtypeStruct((NUM_USERS, D), jnp.float32),
            jax.ShapeDtypeStruct((NUM_POSTS, D), jnp.float32),
        ],
        interpret=interpret,
    )(Cv, Cl, Crv, Crl, xu, xp, wl, wr, bl)


def _pad_edges(ei, pad_dst):
    e = ei.shape[1]
    epad = _ceil_to(e, NSUB * BLK)
    src = jnp.concatenate([ei[0], jnp.zeros((epad - e,), jnp.int32)])
    dst = jnp.concatenate([ei[1], jnp.full((epad - e,), pad_dst, jnp.int32)])
    return src, dst


def kernel(params, emb_user, emb_post, ei_viewed, ei_rev_viewed, ei_liked, ei_rev_liked):
    sv, dv = _pad_edges(ei_viewed, NUM_POSTS)
    sl_, dl_ = _pad_edges(ei_liked, NUM_POSTS)
    srv, drv = _pad_edges(ei_rev_viewed, NUM_USERS)
    srl, drl = _pad_edges(ei_rev_liked, NUM_USERS)

    hv, hl, hrv, hrl = _build_hists(sv, dv, sl_, dl_, srv, drv, srl, drl)
    Cv = hv[:NKEYS].reshape(NUM_POSTS, NUM_USERS)
    Cl = hl[:NKEYS].reshape(NUM_POSTS, NUM_USERS)
    Crv = hrv[:NKEYS].reshape(NUM_USERS, NUM_POSTS)
    Crl = hrl[:NKEYS].reshape(NUM_USERS, NUM_POSTS)

    order = [(1, 'viewed'), (1, 'liked'), (1, 'rev_viewed'), (1, 'rev_liked'),
             (2, 'viewed'), (2, 'liked'), (2, 'rev_viewed'), (2, 'rev_liked')]
    wl = jnp.stack([params['l%d_%s_Wl' % (lyr, rel)] for lyr, rel in order])
    wr = jnp.stack([params['l%d_%s_Wr' % (lyr, rel)] for lyr, rel in order])
    bl = jnp.stack([params['l%d_%s_bl' % (lyr, rel)] for lyr, rel in order])

    u, p = _dense(Cv, Cl, Crv, Crl, emb_user, emb_post, wl, wr, bl)
    return (u, p)


# speedup vs baseline: 104.1731x; 2.2470x over previous
import functools

import jax
import jax.numpy as jnp
from jax import lax
from jax.experimental import pallas as pl
from jax.experimental.pallas import tpu as pltpu
from jax.experimental.pallas import tpu_sc as plsc

NUM_USERS = 631
NUM_POSTS = 1210
D = 64
STRIDE_U = 640
STRIDE_P = 1280
HIST_V = NUM_POSTS * STRIDE_U
HIST_R = NUM_USERS * STRIDE_P
HIST_MAX = max(HIST_V, HIST_R)
NSUB = 16
ZSLICE = HIST_MAX // NSUB
BLK = 1024
NCHUNK = BLK // 128
ZCHUNK = 8192


def _chunks(total):
    return [(o, min(ZCHUNK, total - o)) for o in range(0, total, ZCHUNK)]


def _build_hists(ei_v, ei_l, ei_rv, ei_rl):
    mesh = plsc.VectorSubcoreMesh(core_axis_name="c", subcore_axis_name="s")

    @functools.partial(
        pl.kernel,
        mesh=mesh,
        out_type=[
            jax.ShapeDtypeStruct((HIST_V,), jnp.float32),
            jax.ShapeDtypeStruct((HIST_V,), jnp.float32),
            jax.ShapeDtypeStruct((HIST_R,), jnp.float32),
            jax.ShapeDtypeStruct((HIST_R,), jnp.float32),
        ],
        scratch_types=[
            pltpu.VMEM((BLK,), jnp.int32),
            pltpu.VMEM((BLK,), jnp.int32),
            pltpu.VMEM((BLK,), jnp.int32),
            pltpu.VMEM((BLK,), jnp.int32),
            pltpu.VMEM((BLK,), jnp.int32),
            pltpu.VMEM((BLK,), jnp.int32),
            pltpu.VMEM((128,), jnp.float32),
            pltpu.VMEM((ZCHUNK,), jnp.float32),
            pltpu.VMEM_SHARED((HIST_MAX,), jnp.float32),
            pltpu.VMEM_SHARED((HIST_MAX,), jnp.float32),
            pltpu.SemaphoreType.DMA,
            pltpu.SemaphoreType.DMA,
            pltpu.SemaphoreType.DMA,
            pltpu.SemaphoreType.DMA,
            pltpu.SemaphoreType.DMA,
        ],
    )
    def hist_kernel(ev, el, erv, erl,
                    out_v, out_l, out_rv, out_rl,
                    srcb0, srcb1, dstb0, dstb1, keys0, keys1,
                    vals, stage, hist_a, hist_b,
                    s_ld0, s_ld1, s_sc0, s_sc1, s_io):
        core = lax.axis_index("c")
        sid = lax.axis_index("s")
        s_ld = (s_ld0, s_ld1)
        s_sc = (s_sc0, s_sc1)
        srcb = (srcb0, srcb1)
        dstb = (dstb0, dstb1)
        keys = (keys0, keys1)

        @pl.loop(0, ZCHUNK, step=16)
        def _(i):
            stage[pl.ds(i, 16)] = jnp.zeros((16,), jnp.float32)

        @pl.loop(0, 128, step=16)
        def _(i):
            vals[pl.ds(i, 16)] = jnp.ones((16,), jnp.float32)

        zbase = sid * ZSLICE
        for off, n in _chunks(ZSLICE):
            pltpu.async_copy(stage.at[pl.ds(0, n)], hist_a.at[pl.ds(zbase + off, n)], s_io)
            pltpu.async_copy(stage.at[pl.ds(0, n)], hist_b.at[pl.ds(zbase + off, n)], s_io)
        for off, n in _chunks(ZSLICE):
            pltpu.make_async_copy(stage.at[pl.ds(0, n)], hist_a.at[pl.ds(zbase + off, n)], s_io).wait()
            pltpu.make_async_copy(stage.at[pl.ds(0, n)], hist_b.at[pl.ds(zbase + off, n)], s_io).wait()
        plsc.subcore_barrier()

        def scatter(t, hist, start=False):
            if start:
                for j in range(NCHUNK):
                    pltpu.async_copy(
                        vals, hist.at[keys[t].at[pl.ds(128 * j, 128)]],
                        s_sc[t], add=True)
            else:
                for j in range(NCHUNK):
                    pltpu.make_async_copy(
                        vals, hist.at[keys[t].at[pl.ds(128 * j, 128)]],
                        s_sc[t]).wait()

        def process(edges, mult, hist):
            e = edges.shape[1]
            F = e // BLK
            r = e - F * BLK
            nbmax = -(-F // NSUB)
            nb2 = nbmax + (nbmax & 1)

            def blk_of(ii):
                return sid + NSUB * ii

            def load(ii, t):
                off = blk_of(ii) * BLK
                pltpu.async_copy(edges.at[0, pl.ds(off, BLK)], srcb[t], s_ld[t])
                pltpu.async_copy(edges.at[1, pl.ds(off, BLK)], dstb[t], s_ld[t])

            def wait_load(t):
                pltpu.make_async_copy(edges.at[0, pl.ds(0, BLK)], srcb[t], s_ld[t]).wait()
                pltpu.make_async_copy(edges.at[1, pl.ds(0, BLK)], dstb[t], s_ld[t]).wait()

            def compute_keys(t):
                srow = srcb[t]
                drow = dstb[t]
                krow = keys[t]

                @pl.loop(0, BLK, step=16)
                def _(i):
                    s16 = srow[pl.ds(i, 16)]
                    d16 = drow[pl.ds(i, 16)]
                    krow[pl.ds(i, 16)] = d16 * mult + s16

            @pl.when(blk_of(0) < F)
            def _():
                load(0, 0)

            @pl.when(blk_of(1) < F)
            def _():
                load(1, 1)

            @pl.loop(0, nb2, step=2)
            def _(i):
                for t in (0, 1):
                    ii = i + t

                    @pl.when(blk_of(ii) < F)
                    def _():
                        wait_load(t)

                        @pl.when(blk_of(ii + 2) < F)
                        def _():
                            load(ii + 2, t)

                        @pl.when(ii >= 2)
                        def _():
                            scatter(t, hist)

                        compute_keys(t)
                        scatter(t, hist, start=True)

            for t in (0, 1):
                @pl.when(blk_of(t) < F)
                def _():
                    scatter(t, hist)

            if r:
                assert r % 128 == 0 and r < BLK

                @pl.when(sid == NSUB - 1)
                def _():
                    pltpu.sync_copy(edges.at[0, pl.ds(F * BLK, r)],
                                    srcb[0].at[pl.ds(0, r)])
                    pltpu.sync_copy(edges.at[1, pl.ds(F * BLK, r)],
                                    dstb[0].at[pl.ds(0, r)])
                    srow = srcb[0]
                    drow = dstb[0]
                    krow = keys[0]

                    @pl.loop(0, r, step=16)
                    def _(i):
                        s16 = srow[pl.ds(i, 16)]
                        d16 = drow[pl.ds(i, 16)]
                        krow[pl.ds(i, 16)] = d16 * mult + s16

                    for j in range(r // 128):
                        pltpu.sync_copy(
                            vals, hist.at[keys[0].at[pl.ds(128 * j, 128)]],
                            add=True)

        @pl.when(core == 0)
        def _():
            process(ev, STRIDE_U, hist_a)
            process(el, STRIDE_U, hist_b)

        @pl.when(core == 1)
        def _():
            process(erv, STRIDE_P, hist_a)
            process(erl, STRIDE_P, hist_b)

        plsc.subcore_barrier()

        def readout(hist, out, hist_n):
            rslice = hist_n // NSUB
            base = sid * rslice
            for off, n in _chunks(rslice):
                pltpu.sync_copy(hist.at[pl.ds(base + off, n)], stage.at[pl.ds(0, n)])
                pltpu.sync_copy(stage.at[pl.ds(0, n)], out.at[pl.ds(base + off, n)])

        @pl.when(core == 0)
        def _():
            readout(hist_a, out_v, HIST_V)
            readout(hist_b, out_l, HIST_V)

        @pl.when(core == 1)
        def _():
            readout(hist_a, out_rv, HIST_R)
            readout(hist_b, out_rl, HIST_R)

    return hist_kernel(ei_v, ei_l, ei_rv, ei_rl)


def _dense_body(cv, cl, crv, crl, xu, xp, wl, wr, bl, u_out, p_out):
    Cv = cv[...]
    Cl = cl[...]
    Crv = crv[...]
    Crl = crl[...]
    xu_ = xu[...]
    xp_ = xp[...]

    sel_u = (lax.broadcasted_iota(jnp.int32, (STRIDE_U, 1), 0) < NUM_USERS
             ).astype(jnp.float32)
    sel_p = (lax.broadcasted_iota(jnp.int32, (STRIDE_P, 1), 0) < NUM_POSTS
             ).astype(jnp.float32)
    inv_v = 1.0 / jnp.maximum(jnp.dot(Cv, sel_u, preferred_element_type=jnp.float32), 1.0)
    inv_l = 1.0 / jnp.maximum(jnp.dot(Cl, sel_u, preferred_element_type=jnp.float32), 1.0)
    inv_rv = 1.0 / jnp.maximum(jnp.dot(Crv, sel_p, preferred_element_type=jnp.float32), 1.0)
    inv_rl = 1.0 / jnp.maximum(jnp.dot(Crl, sel_p, preferred_element_type=jnp.float32), 1.0)

    zu = jnp.zeros((STRIDE_U - NUM_USERS, D), jnp.float32)
    zp = jnp.zeros((STRIDE_P - NUM_POSTS, D), jnp.float32)

    def conv(C, inv, xs_pad, xd, i):
        mean = jnp.dot(C, xs_pad, preferred_element_type=jnp.float32) * inv
        return (jnp.dot(mean, wl[i], preferred_element_type=jnp.float32)
                + bl[i][None, :]
                + jnp.dot(xd, wr[i], preferred_element_type=jnp.float32))

    xu_pad = jnp.concatenate([xu_, zu], axis=0)
    xp_pad = jnp.concatenate([xp_, zp], axis=0)
    p1 = jax.nn.relu(conv(Cv, inv_v, xu_pad, xp_, 0) + conv(Cl, inv_l, xu_pad, xp_, 1))
    u1 = jax.nn.relu(conv(Crv, inv_rv, xp_pad, xu_, 2) + conv(Crl, inv_rl, xp_pad, xu_, 3))
    u1_pad = jnp.concatenate([u1, zu], axis=0)
    p1_pad = jnp.concatenate([p1, zp], axis=0)
    p2 = conv(Cv, inv_v, u1_pad, p1, 4) + conv(Cl, inv_l, u1_pad, p1, 5)
    u2 = conv(Crv, inv_rv, p1_pad, u1, 6) + conv(Crl, inv_rl, p1_pad, u1, 7)
    u_out[...] = u2
    p_out[...] = p2


def _dense(Cv, Cl, Crv, Crl, xu, xp, wl, wr, bl, interpret=False):
    return pl.pallas_call(
        _dense_body,
        out_shape=[
            jax.ShapeDtypeStruct((NUM_USERS, D), jnp.float32),
            jax.ShapeDtypeStruct((NUM_POSTS, D), jnp.float32),
        ],
        interpret=interpret,
    )(Cv, Cl, Crv, Crl, xu, xp, wl, wr, bl)


def _pad_edges(e, dead):
    n = e.shape[1]
    pad = -n % 128
    if not pad:
        return e
    fill = jnp.broadcast_to(jnp.array([[dead], [0]], jnp.int32), (2, pad))
    return jnp.concatenate([e, fill], axis=1)


def kernel(params, emb_user, emb_post, ei_viewed, ei_rev_viewed, ei_liked, ei_rev_liked):
    ei_viewed = _pad_edges(ei_viewed, NUM_USERS)
    ei_liked = _pad_edges(ei_liked, NUM_USERS)
    ei_rev_viewed = _pad_edges(ei_rev_viewed, NUM_POSTS)
    ei_rev_liked = _pad_edges(ei_rev_liked, NUM_POSTS)
    hv, hl, hrv, hrl = _build_hists(ei_viewed, ei_liked, ei_rev_viewed, ei_rev_liked)
    Cv = hv.reshape(NUM_POSTS, STRIDE_U)
    Cl = hl.reshape(NUM_POSTS, STRIDE_U)
    Crv = hrv.reshape(NUM_USERS, STRIDE_P)
    Crl = hrl.reshape(NUM_USERS, STRIDE_P)

    order = [(1, 'viewed'), (1, 'liked'), (1, 'rev_viewed'), (1, 'rev_liked'),
             (2, 'viewed'), (2, 'liked'), (2, 'rev_viewed'), (2, 'rev_liked')]
    wl = jnp.stack([params['l%d_%s_Wl' % (lyr, rel)] for lyr, rel in order])
    wr = jnp.stack([params['l%d_%s_Wr' % (lyr, rel)] for lyr, rel in order])
    bl = jnp.stack([params['l%d_%s_bl' % (lyr, rel)] for lyr, rel in order])

    u, p = _dense(Cv, Cl, Crv, Crl, emb_user, emb_post, wl, wr, bl)
    return (u, p)


# speedup vs baseline: 124.3137x; 1.1933x over previous
import functools

import jax
import jax.numpy as jnp
from jax import lax
from jax.experimental import pallas as pl
from jax.experimental.pallas import tpu as pltpu
from jax.experimental.pallas import tpu_sc as plsc

NUM_USERS = 631
NUM_POSTS = 1210
D = 64
SBLK_U = 5
SBLK_P = 10
ROWS_P = 1216
ROWS_U = 632
BAND_U = ROWS_P * 128
BAND_P = ROWS_U * 128
HIST_V = SBLK_U * BAND_U
HIST_R = SBLK_P * BAND_P
HIST_MAX = max(HIST_V, HIST_R)
NSUB = 16
ZSLICE = HIST_MAX // NSUB
BLK = 1024
NCHUNK = BLK // 128
ZCHUNK = 8192


def _chunks(total):
    return [(o, min(ZCHUNK, total - o)) for o in range(0, total, ZCHUNK)]


def _build_hists(ei_v, ei_l, ei_rv, ei_rl, tl_v, tl_l, tl_rv, tl_rl):
    mesh = plsc.VectorSubcoreMesh(core_axis_name="c", subcore_axis_name="s")

    @functools.partial(
        pl.kernel,
        mesh=mesh,
        out_type=[
            jax.ShapeDtypeStruct((HIST_V,), jnp.float32),
            jax.ShapeDtypeStruct((HIST_V,), jnp.float32),
            jax.ShapeDtypeStruct((HIST_R,), jnp.float32),
            jax.ShapeDtypeStruct((HIST_R,), jnp.float32),
        ],
        scratch_types=[
            pltpu.VMEM((2, BLK), jnp.int32),
            pltpu.VMEM((2, BLK), jnp.int32),
            pltpu.VMEM((BLK,), jnp.int32),
            pltpu.VMEM((BLK,), jnp.int32),
            pltpu.VMEM((128,), jnp.float32),
            pltpu.VMEM((ZCHUNK,), jnp.float32),
            pltpu.VMEM_SHARED((HIST_MAX,), jnp.float32),
            pltpu.VMEM_SHARED((HIST_MAX,), jnp.float32),
            pltpu.SemaphoreType.DMA,
            pltpu.SemaphoreType.DMA,
            pltpu.SemaphoreType.DMA,
            pltpu.SemaphoreType.DMA,
            pltpu.SemaphoreType.DMA,
        ],
    )
    def hist_kernel(ev, el, erv, erl, tv, tl, trv, trl,
                    out_v, out_l, out_rv, out_rl,
                    edb0, edb1, keys0, keys1,
                    vals, stage, hist_a, hist_b,
                    s_ld0, s_ld1, s_sc0, s_sc1, s_io):
        core = lax.axis_index("c")
        sid = lax.axis_index("s")
        s_ld = (s_ld0, s_ld1)
        s_sc = (s_sc0, s_sc1)
        edb = (edb0, edb1)
        keys = (keys0, keys1)

        @pl.loop(0, ZCHUNK, step=16)
        def _(i):
            stage[pl.ds(i, 16)] = jnp.zeros((16,), jnp.float32)

        @pl.loop(0, 128, step=16)
        def _(i):
            vals[pl.ds(i, 16)] = jnp.ones((16,), jnp.float32)

        zbase = sid * ZSLICE
        for off, n in _chunks(ZSLICE):
            pltpu.async_copy(stage.at[pl.ds(0, n)], hist_a.at[pl.ds(zbase + off, n)], s_io)
            pltpu.async_copy(stage.at[pl.ds(0, n)], hist_b.at[pl.ds(zbase + off, n)], s_io)
        for off, n in _chunks(ZSLICE):
            pltpu.make_async_copy(stage.at[pl.ds(0, n)], hist_a.at[pl.ds(zbase + off, n)], s_io).wait()
            pltpu.make_async_copy(stage.at[pl.ds(0, n)], hist_b.at[pl.ds(zbase + off, n)], s_io).wait()
        plsc.subcore_barrier()

        def scatter(t, hist, start=False):
            if start:
                for j in range(NCHUNK):
                    pltpu.async_copy(
                        vals, hist.at[keys[t].at[pl.ds(128 * j, 128)]],
                        s_sc[t], add=True)
            else:
                for j in range(NCHUNK):
                    pltpu.make_async_copy(
                        vals, hist.at[keys[t].at[pl.ds(128 * j, 128)]],
                        s_sc[t]).wait()

        def process(edges, tail, band, hist):
            e = edges.shape[1]
            F = e // BLK
            r = e - F * BLK
            nbmax = -(-F // NSUB)
            nb2 = nbmax + (nbmax & 1)

            def blk_of(ii):
                return sid + NSUB * ii

            def load(ii, t):
                off = blk_of(ii) * BLK
                pltpu.async_copy(edges.at[:, pl.ds(off, BLK)], edb[t], s_ld[t])

            def wait_load(t):
                pltpu.make_async_copy(edges.at[:, pl.ds(0, BLK)], edb[t], s_ld[t]).wait()

            def compute_keys(t):
                srow = edb[t].at[0]
                drow = edb[t].at[1]
                krow = keys[t]

                @pl.loop(0, BLK, step=16)
                def _(i):
                    s16 = srow[pl.ds(i, 16)]
                    d16 = drow[pl.ds(i, 16)]
                    krow[pl.ds(i, 16)] = ((s16 >> 7) * band + (d16 << 7)
                                          + (s16 & 127))

            @pl.when(blk_of(0) < F)
            def _():
                load(0, 0)

            @pl.when(blk_of(1) < F)
            def _():
                load(1, 1)

            @pl.loop(0, nb2, step=2)
            def _(i):
                for t in (0, 1):
                    ii = i + t

                    @pl.when(blk_of(ii) < F)
                    def _():
                        wait_load(t)

                        @pl.when(blk_of(ii + 2) < F)
                        def _():
                            load(ii + 2, t)

                        @pl.when(ii >= 2)
                        def _():
                            scatter(t, hist)

                        compute_keys(t)
                        scatter(t, hist, start=True)

            for t in (0, 1):
                @pl.when(blk_of(t) < F)
                def _():
                    scatter(t, hist)

            if r:
                @pl.when(sid == NSUB - 1)
                def _():
                    pltpu.sync_copy(tail, edb[0])
                    compute_keys(0)
                    scatter(0, hist, start=True)
                    scatter(0, hist)

        @pl.when(core == 0)
        def _():
            process(ev, tv, BAND_U, hist_a)
            process(el, tl, BAND_U, hist_b)

        @pl.when(core == 1)
        def _():
            process(erv, trv, BAND_P, hist_a)
            process(erl, trl, BAND_P, hist_b)

        plsc.subcore_barrier()

        def readout(hist, out, hist_n):
            rslice = hist_n // NSUB
            base = sid * rslice
            for off, n in _chunks(rslice):
                pltpu.sync_copy(hist.at[pl.ds(base + off, n)], stage.at[pl.ds(0, n)])
                pltpu.sync_copy(stage.at[pl.ds(0, n)], out.at[pl.ds(base + off, n)])

        @pl.when(core == 0)
        def _():
            readout(hist_a, out_v, HIST_V)
            readout(hist_b, out_l, HIST_V)

        @pl.when(core == 1)
        def _():
            readout(hist_a, out_rv, HIST_R)
            readout(hist_b, out_rl, HIST_R)

    return hist_kernel(ei_v, ei_l, ei_rv, ei_rl, tl_v, tl_l, tl_rv, tl_rl)


def _dot(a, b):
    return jnp.dot(a, b, preferred_element_type=jnp.float32)


def _dense_body(cv, cl, crv, crl, xu, xp, wl, wr, bl, u_out, p_out):
    Cv = cv[...].reshape(SBLK_U * ROWS_P, 128)
    Cl = cl[...].reshape(SBLK_U * ROWS_P, 128)
    Crv = crv[...].reshape(SBLK_P * ROWS_U, 128)
    Crl = crl[...].reshape(SBLK_P * ROWS_U, 128)
    xu_ = xu[...]
    xp_ = xp[...]

    def blocks(C, rows, nb):
        return [C[b * rows:(b + 1) * rows] for b in range(nb)]

    Cvb = blocks(Cv, ROWS_P, SBLK_U)
    Clb = blocks(Cl, ROWS_P, SBLK_U)
    Crvb = blocks(Crv, ROWS_U, SBLK_P)
    Crlb = blocks(Crl, ROWS_U, SBLK_P)

    lane = lax.broadcasted_iota(jnp.int32, (128, 1), 0)

    def inv_cnt(Cb, n_src, n_dst):
        cnt = 0.0
        for b, C in enumerate(Cb):
            sel = (lane + 128 * b < n_src).astype(jnp.float32)
            cnt = cnt + _dot(C, sel)
        return (1.0 / jnp.maximum(cnt, 1.0))[:n_dst]

    inv_v = inv_cnt(Cvb, NUM_USERS, NUM_POSTS)
    inv_l = inv_cnt(Clb, NUM_USERS, NUM_POSTS)
    inv_rv = inv_cnt(Crvb, NUM_POSTS, NUM_USERS)
    inv_rl = inv_cnt(Crlb, NUM_POSTS, NUM_USERS)

    def conv(Cb, inv, xs_pad, xd, n_dst, i):
        sums = 0.0
        for b, C in enumerate(Cb):
            sums = sums + _dot(C, xs_pad[128 * b:128 * (b + 1)])
        mean = sums[:n_dst] * inv
        return (_dot(mean, wl[i]) + bl[i][None, :] + _dot(xd, wr[i]))

    zu = jnp.zeros((SBLK_U * 128 - NUM_USERS, D), jnp.float32)
    zp = jnp.zeros((SBLK_P * 128 - NUM_POSTS, D), jnp.float32)

    xu_pad = jnp.concatenate([xu_, zu], axis=0)
    xp_pad = jnp.concatenate([xp_, zp], axis=0)
    p1 = jax.nn.relu(conv(Cvb, inv_v, xu_pad, xp_, NUM_POSTS, 0)
                     + conv(Clb, inv_l, xu_pad, xp_, NUM_POSTS, 1))
    u1 = jax.nn.relu(conv(Crvb, inv_rv, xp_pad, xu_, NUM_USERS, 2)
                     + conv(Crlb, inv_rl, xp_pad, xu_, NUM_USERS, 3))
    u1_pad = jnp.concatenate([u1, zu], axis=0)
    p1_pad = jnp.concatenate([p1, zp], axis=0)
    p2 = conv(Cvb, inv_v, u1_pad, p1, NUM_POSTS, 4) + conv(Clb, inv_l, u1_pad, p1, NUM_POSTS, 5)
    u2 = conv(Crvb, inv_rv, p1_pad, u1, NUM_USERS, 6) + conv(Crlb, inv_rl, p1_pad, u1, NUM_USERS, 7)
    u_out[...] = u2
    p_out[...] = p2


def _dense(Cv, Cl, Crv, Crl, xu, xp, wl, wr, bl, interpret=False):
    return pl.pallas_call(
        _dense_body,
        out_shape=[
            jax.ShapeDtypeStruct((NUM_USERS, D), jnp.float32),
            jax.ShapeDtypeStruct((NUM_POSTS, D), jnp.float32),
        ],
        interpret=interpret,
    )(Cv, Cl, Crv, Crl, xu, xp, wl, wr, bl)


def _tail_block(e, dead):
    n = e.shape[1]
    r = n % BLK
    fill = jnp.broadcast_to(jnp.array([[dead], [0]], jnp.int32), (2, BLK - r))
    return jnp.concatenate([e[:, n - r:], fill], axis=1)


def kernel(params, emb_user, emb_post, ei_viewed, ei_rev_viewed, ei_liked, ei_rev_liked):
    tv = _tail_block(ei_viewed, NUM_USERS)
    tl = _tail_block(ei_liked, NUM_USERS)
    trv = _tail_block(ei_rev_viewed, NUM_POSTS)
    trl = _tail_block(ei_rev_liked, NUM_POSTS)
    Cv, Cl, Crv, Crl = _build_hists(
        ei_viewed, ei_liked, ei_rev_viewed, ei_rev_liked, tv, tl, trv, trl)

    order = [(1, 'viewed'), (1, 'liked'), (1, 'rev_viewed'), (1, 'rev_liked'),
             (2, 'viewed'), (2, 'liked'), (2, 'rev_viewed'), (2, 'rev_liked')]
    wl = jnp.stack([params['l%d_%s_Wl' % (lyr, rel)] for lyr, rel in order])
    wr = jnp.stack([params['l%d_%s_Wr' % (lyr, rel)] for lyr, rel in order])
    bl = jnp.stack([params['l%d_%s_bl' % (lyr, rel)] for lyr, rel in order])

    u, p = _dense(Cv, Cl, Crv, Crl, emb_user, emb_post, wl, wr, bl)
    return (u, p)


# speedup vs baseline: 128.1534x; 1.0309x over previous
import functools

import jax
import jax.numpy as jnp
from jax import lax
from jax.experimental import pallas as pl
from jax.experimental.pallas import tpu as pltpu
from jax.experimental.pallas import tpu_sc as plsc

NUM_USERS = 631
NUM_POSTS = 1210
D = 64
SBLK_U = 5
SBLK_P = 10
ROWS_P = 1216
ROWS_U = 632
BAND_U = ROWS_P * 128
BAND_P = ROWS_U * 128
HIST_V = SBLK_U * BAND_U
HIST_R = SBLK_P * BAND_P
HIST_MAX = max(HIST_V, HIST_R)
NSUB = 16
ZSLICE = HIST_MAX // NSUB
BLK = 1024
NCHUNK = BLK // 128
ZCHUNK = 8192


def _chunks(total):
    return [(o, min(ZCHUNK, total - o)) for o in range(0, total, ZCHUNK)]


def _build_hists(ei_v, ei_l, ei_rv, ei_rl, tl_v, tl_l, tl_rv, tl_rl):
    mesh = plsc.VectorSubcoreMesh(core_axis_name="c", subcore_axis_name="s")

    @functools.partial(
        pl.kernel,
        mesh=mesh,
        out_type=[
            jax.ShapeDtypeStruct((HIST_V,), jnp.float32),
            jax.ShapeDtypeStruct((HIST_V,), jnp.float32),
            jax.ShapeDtypeStruct((HIST_R,), jnp.float32),
            jax.ShapeDtypeStruct((HIST_R,), jnp.float32),
        ],
        scratch_types=[
            pltpu.VMEM((2, BLK), jnp.int32),
            pltpu.VMEM((2, BLK), jnp.int32),
            pltpu.VMEM((BLK,), jnp.int32),
            pltpu.VMEM((BLK,), jnp.int32),
            pltpu.VMEM((128,), jnp.float32),
            pltpu.VMEM((ZCHUNK,), jnp.float32),
            pltpu.VMEM((ZCHUNK,), jnp.float32),
            pltpu.VMEM_SHARED((HIST_MAX,), jnp.float32),
            pltpu.VMEM_SHARED((HIST_MAX,), jnp.float32),
            pltpu.SemaphoreType.DMA,
            pltpu.SemaphoreType.DMA,
            pltpu.SemaphoreType.DMA,
            pltpu.SemaphoreType.DMA,
            pltpu.SemaphoreType.DMA,
            pltpu.SemaphoreType.DMA,
        ],
    )
    def hist_kernel(ev, el, erv, erl, tv, tl, trv, trl,
                    out_v, out_l, out_rv, out_rl,
                    edb0, edb1, keys0, keys1,
                    vals, stage, stage1, hist_a, hist_b,
                    s_ld0, s_ld1, s_sc0, s_sc1, s_io, s_io2):
        core = lax.axis_index("c")
        sid = lax.axis_index("s")
        s_ld = (s_ld0, s_ld1)
        s_sc = (s_sc0, s_sc1)
        edb = (edb0, edb1)
        keys = (keys0, keys1)

        @pl.loop(0, ZCHUNK, step=16)
        def _(i):
            stage[pl.ds(i, 16)] = jnp.zeros((16,), jnp.float32)

        @pl.loop(0, 128, step=16)
        def _(i):
            vals[pl.ds(i, 16)] = jnp.ones((16,), jnp.float32)

        zbase = sid * ZSLICE
        for off, n in _chunks(ZSLICE):
            pltpu.async_copy(stage.at[pl.ds(0, n)], hist_a.at[pl.ds(zbase + off, n)], s_io)
            pltpu.async_copy(stage.at[pl.ds(0, n)], hist_b.at[pl.ds(zbase + off, n)], s_io2)
        for off, n in _chunks(ZSLICE):
            pltpu.make_async_copy(stage.at[pl.ds(0, n)], hist_a.at[pl.ds(zbase + off, n)], s_io).wait()
        plsc.subcore_barrier()

        def scatter(t, hist, start=False):
            if start:
                for j in range(NCHUNK):
                    pltpu.async_copy(
                        vals, hist.at[keys[t].at[pl.ds(128 * j, 128)]],
                        s_sc[t], add=True)
            else:
                for j in range(NCHUNK):
                    pltpu.make_async_copy(
                        vals, hist.at[keys[t].at[pl.ds(128 * j, 128)]],
                        s_sc[t]).wait()

        def process(edges, tail, band, hist):
            e = edges.shape[1]
            F = e // BLK
            r = e - F * BLK
            nbmax = -(-F // NSUB)
            nb2 = nbmax + (nbmax & 1)

            def blk_of(ii):
                return sid + NSUB * ii

            def load(ii, t):
                off = blk_of(ii) * BLK
                pltpu.async_copy(edges.at[:, pl.ds(off, BLK)], edb[t], s_ld[t])

            def wait_load(t):
                pltpu.make_async_copy(edges.at[:, pl.ds(0, BLK)], edb[t], s_ld[t]).wait()

            def compute_keys(t):
                srow = edb[t].at[0]
                drow = edb[t].at[1]
                krow = keys[t]

                @pl.loop(0, BLK, step=16)
                def _(i):
                    s16 = srow[pl.ds(i, 16)]
                    d16 = drow[pl.ds(i, 16)]
                    krow[pl.ds(i, 16)] = (s16 + (s16 >> 7) * (band - 128)
                                          + (d16 << 7))

            @pl.when(blk_of(0) < F)
            def _():
                load(0, 0)

            @pl.when(blk_of(1) < F)
            def _():
                load(1, 1)

            @pl.loop(0, nb2, step=2)
            def _(i):
                for t in (0, 1):
                    ii = i + t

                    @pl.when(blk_of(ii) < F)
                    def _():
                        wait_load(t)

                        @pl.when(ii >= 2)
                        def _():
                            scatter(t, hist)

                        compute_keys(t)

                        @pl.when(blk_of(ii + 2) < F)
                        def _():
                            load(ii + 2, t)

                        scatter(t, hist, start=True)

            for t in (0, 1):
                @pl.when(blk_of(t) < F)
                def _():
                    scatter(t, hist)

            if r:
                @pl.when(sid == NSUB - 1)
                def _():
                    pltpu.sync_copy(tail, edb[0])
                    compute_keys(0)
                    scatter(0, hist, start=True)
                    scatter(0, hist)

        @pl.when(core == 0)
        def _():
            process(ev, tv, BAND_U, hist_a)

        @pl.when(core == 1)
        def _():
            process(erv, trv, BAND_P, hist_a)

        for off, n in _chunks(ZSLICE):
            pltpu.make_async_copy(stage.at[pl.ds(0, n)], hist_b.at[pl.ds(zbase + off, n)], s_io2).wait()
        plsc.subcore_barrier()

        @pl.when(core == 0)
        def _():
            process(el, tl, BAND_U, hist_b)

        @pl.when(core == 1)
        def _():
            process(erl, trl, BAND_P, hist_b)

        plsc.subcore_barrier()

        def readout(hist, out, hist_n, chain):
            rslice = hist_n // NSUB
            base = sid * rslice
            stg = (stage, stage1)
            sem = (s_io, s_io2)
            for off, n in _chunks(rslice):
                t = len(chain) & 1
                if len(chain) >= 2:
                    chain[-2]().wait()
                pltpu.sync_copy(hist.at[pl.ds(base + off, n)], stg[t].at[pl.ds(0, n)])
                src = stg[t].at[pl.ds(0, n)]
                dst = out.at[pl.ds(base + off, n)]
                pltpu.async_copy(src, dst, sem[t])
                chain.append(functools.partial(pltpu.make_async_copy, src, dst, sem[t]))

        def drain(chain):
            for mk in chain[-2:]:
                mk().wait()

        @pl.when(core == 0)
        def _():
            chain = []
            readout(hist_a, out_v, HIST_V, chain)
            readout(hist_b, out_l, HIST_V, chain)
            drain(chain)

        @pl.when(core == 1)
        def _():
            chain = []
            readout(hist_a, out_rv, HIST_R, chain)
            readout(hist_b, out_rl, HIST_R, chain)
            drain(chain)

    return hist_kernel(ei_v, ei_l, ei_rv, ei_rl, tl_v, tl_l, tl_rv, tl_rl)


def _dot(a, b):
    return jnp.dot(a, b, preferred_element_type=jnp.float32)


def _dense_body(cv, cl, crv, crl, xu, xp, wl, wr, bl, u_out, p_out):
    Cv = cv[...].reshape(SBLK_U * ROWS_P, 128)
    Cl = cl[...].reshape(SBLK_U * ROWS_P, 128)
    Crv = crv[...].reshape(SBLK_P * ROWS_U, 128)
    Crl = crl[...].reshape(SBLK_P * ROWS_U, 128)
    xu_ = xu[...]
    xp_ = xp[...]

    def blocks(C, rows, nb):
        return [C[b * rows:(b + 1) * rows] for b in range(nb)]

    Cvb = blocks(Cv, ROWS_P, SBLK_U)
    Clb = blocks(Cl, ROWS_P, SBLK_U)
    Crvb = blocks(Crv, ROWS_U, SBLK_P)
    Crlb = blocks(Crl, ROWS_U, SBLK_P)

    def aug(x, n_src, nblk):
        rows = nblk * 128
        sel = (lax.broadcasted_iota(jnp.int32, (rows, 1), 0) < n_src
               ).astype(jnp.float32)
        z = jnp.zeros((rows - x.shape[0], D), jnp.float32)
        xp_full = jnp.concatenate([x, z], axis=0)
        x_hi = xp_full.astype(jnp.bfloat16).astype(jnp.float32)
        x_lo = xp_full - x_hi
        return jnp.concatenate([x_hi, sel, x_lo], axis=1)

    def agg(Cb, x_aug, n_dst):
        acc = 0.0
        for b, C in enumerate(Cb):
            acc = acc + _dot(C, x_aug[128 * b:128 * (b + 1)])
        sums = acc[:n_dst, :D] + acc[:n_dst, D + 1:2 * D + 1]
        cnt = acc[:n_dst, D:D + 1]
        return sums, cnt

    def mean(sums, cnt):
        return sums / jnp.maximum(cnt, 1.0)

    def head(mean_a, mean_b, xd, ia, ib):
        return ((_dot(mean_a, wl[ia]) + bl[ia][None, :] + _dot(xd, wr[ia]))
                + (_dot(mean_b, wl[ib]) + bl[ib][None, :] + _dot(xd, wr[ib])))

    xu_aug = aug(xu_, NUM_USERS, SBLK_U)
    xp_aug = aug(xp_, NUM_POSTS, SBLK_P)
    sums_v, cnt_v = agg(Cvb, xu_aug, NUM_POSTS)
    sums_l, cnt_l = agg(Clb, xu_aug, NUM_POSTS)
    sums_rv, cnt_rv = agg(Crvb, xp_aug, NUM_USERS)
    sums_rl, cnt_rl = agg(Crlb, xp_aug, NUM_USERS)

    p1 = jax.nn.relu(head(mean(sums_v, cnt_v), mean(sums_l, cnt_l), xp_, 0, 1))
    u1 = jax.nn.relu(head(mean(sums_rv, cnt_rv), mean(sums_rl, cnt_rl), xu_, 2, 3))

    u1_aug = aug(u1, NUM_USERS, SBLK_U)
    p1_aug = aug(p1, NUM_POSTS, SBLK_P)
    sums_v2, _ = agg(Cvb, u1_aug, NUM_POSTS)
    sums_l2, _ = agg(Clb, u1_aug, NUM_POSTS)
    sums_rv2, _ = agg(Crvb, p1_aug, NUM_USERS)
    sums_rl2, _ = agg(Crlb, p1_aug, NUM_USERS)
    p2 = head(mean(sums_v2, cnt_v), mean(sums_l2, cnt_l), p1, 4, 5)
    u2 = head(mean(sums_rv2, cnt_rv), mean(sums_rl2, cnt_rl), u1, 6, 7)
    u_out[...] = u2
    p_out[...] = p2


def _dense(Cv, Cl, Crv, Crl, xu, xp, wl, wr, bl, interpret=False):
    return pl.pallas_call(
        _dense_body,
        out_shape=[
            jax.ShapeDtypeStruct((NUM_USERS, D), jnp.float32),
            jax.ShapeDtypeStruct((NUM_POSTS, D), jnp.float32),
        ],
        interpret=interpret,
    )(Cv, Cl, Crv, Crl, xu, xp, wl, wr, bl)


def _tail_block(e, dead):
    n = e.shape[1]
    r = n % BLK
    fill = jnp.broadcast_to(jnp.array([[dead], [0]], jnp.int32), (2, BLK - r))
    return jnp.concatenate([e[:, n - r:], fill], axis=1)


def kernel(params, emb_user, emb_post, ei_viewed, ei_rev_viewed, ei_liked, ei_rev_liked):
    tv = _tail_block(ei_viewed, NUM_USERS)
    tl = _tail_block(ei_liked, NUM_USERS)
    trv = _tail_block(ei_rev_viewed, NUM_POSTS)
    trl = _tail_block(ei_rev_liked, NUM_POSTS)
    Cv, Cl, Crv, Crl = _build_hists(
        ei_viewed, ei_liked, ei_rev_viewed, ei_rev_liked, tv, tl, trv, trl)

    order = [(1, 'viewed'), (1, 'liked'), (1, 'rev_viewed'), (1, 'rev_liked'),
             (2, 'viewed'), (2, 'liked'), (2, 'rev_viewed'), (2, 'rev_liked')]
    wl = jnp.stack([params['l%d_%s_Wl' % (lyr, rel)] for lyr, rel in order])
    wr = jnp.stack([params['l%d_%s_Wr' % (lyr, rel)] for lyr, rel in order])
    bl = jnp.stack([params['l%d_%s_bl' % (lyr, rel)] for lyr, rel in order])

    u, p = _dense(Cv, Cl, Crv, Crl, emb_user, emb_post, wl, wr, bl)
    return (u, p)
